# baseline (speedup 1.0000x reference)
import functools

import jax
import jax.numpy as jnp
from jax import lax
from jax.experimental import pallas as pl
from jax.experimental.pallas import tpu as pltpu
from jax.experimental.pallas import tpu_sc as plsc

K = 3
BN = 512
CH = 32
LANES = 16


def _topk_block(p_ref, pb_ref, vt_ref, vtb_ref, idx_ref, w_ref, *, m):
    p = p_ref[...]
    pb = pb_ref[...]
    vt = vt_ref[...]
    vtb = vtb_ref[...]
    dot = jnp.dot(pb, vtb, preferred_element_type=jnp.float32)
    x, y, z = p[:, 0:1], p[:, 1:2], p[:, 2:3]
    p2 = x * x + y * y + z * z
    v0, v1, v2 = vt[0:1, :], vt[1:2, :], vt[2:3, :]
    vsq = v0 * v0 + v1 * v1 + v2 * v2
    pvsum = p2 + vsq
    dmat = pvsum - 2.0 * dot

    colsf = lax.broadcasted_iota(jnp.int32, dmat.shape, 1).astype(jnp.float32)
    mf = jnp.float32(m)
    inf = jnp.float32(jnp.inf)
    dw = dmat
    ikfs, ohs = [], []
    for k in range(K):
        dk = jnp.min(dw, axis=1, keepdims=True)
        ikf = jnp.min(jnp.where(dw == dk, colsf, mf), axis=1, keepdims=True)
        oh = colsf == ikf
        if k < K - 1:
            dw = jnp.where(oh, inf, dw)
        ikfs.append(ikf)
        ohs.append(oh)

    dotf = x * v0 + y * v1 + z * v2
    zero = jnp.float32(0.0)
    ddir = jnp.maximum(pvsum - 2.0 * dotf, zero)
    recips = [1.0 / (jnp.sum(jnp.where(oh, ddir, zero), axis=1, keepdims=True)
                     + 1e-8) for oh in ohs]
    norm = recips[0] + recips[1] + recips[2]
    idx_ref[...] = jnp.concatenate(
        [ikf.astype(jnp.int32) for ikf in ikfs], axis=1)
    w_ref[...] = jnp.concatenate(
        [jnp.broadcast_to(rk / norm, (p.shape[0], LANES)) for rk in recips],
        axis=1)


def _sc_interp(vf_hbm, idx_hbm, w_hbm, out_hbm, idx_v, rows_v, w_v, out_buf,
               sem, *, q_per_w, n_chunks):
    info = plsc.get_sparse_core_info()
    nc = info.num_cores
    wid = lax.axis_index("s") * nc + lax.axis_index("c")
    qbase = wid * q_per_w

    def chunk_body(c, carry):
        rbase = (qbase + c * CH) * K
        pltpu.sync_copy(idx_hbm.at[pl.ds(rbase, CH * K)], idx_v)
        pltpu.async_copy(vf_hbm.at[idx_v], rows_v, sem).wait()
        pltpu.sync_copy(w_hbm.at[pl.ds(rbase, CH * K)], w_v)

        def q_body(qi, qcarry):
            r0 = qi * K
            w0 = w_v[r0, :]
            w1 = w_v[r0 + 1, :]
            w2 = w_v[r0 + 2, :]
            for j in range(0, 256, LANES):
                seg = pl.ds(j, LANES)
                acc = w0 * rows_v[r0, seg]
                acc = acc + w1 * rows_v[r0 + 1, seg]
                acc = acc + w2 * rows_v[r0 + 2, seg]
                out_buf[qi, seg] = acc
            return qcarry

        lax.fori_loop(0, CH, q_body, 0)
        pltpu.sync_copy(out_buf, out_hbm.at[pl.ds(qbase + c * CH, CH)])
        return carry

    lax.fori_loop(0, n_chunks, chunk_body, 0)


NSLICE = 16


@jax.jit
def kernel(v_coor, v_feats, p_coor):
    n, m = p_coor.shape[0], v_coor.shape[0]
    d = v_feats.shape[1]
    p_pad = jnp.pad(p_coor, ((0, 0), (0, 5)))
    vt_pad = jnp.pad(v_coor.T, ((0, 5), (0, 0)))
    pb_pad = p_pad.astype(jnp.bfloat16)
    vtb_pad = vt_pad.astype(jnp.bfloat16)

    info = plsc.get_sparse_core_info()
    n_workers = info.num_cores * info.num_subcores
    ns = n // NSLICE
    q_per_w = ns // n_workers
    n_chunks = q_per_w // CH
    mesh = plsc.VectorSubcoreMesh(core_axis_name="c", subcore_axis_name="s")
    sc = functools.partial(
        pl.kernel,
        mesh=mesh,
        out_type=jax.ShapeDtypeStruct((ns, d), jnp.float32),
        scratch_types=[
            pltpu.VMEM((CH * K,), jnp.int32),
            pltpu.VMEM((CH * K, d), jnp.float32),
            pltpu.VMEM((CH * K, LANES), jnp.float32),
            pltpu.VMEM((CH, d), jnp.float32),
            pltpu.SemaphoreType.DMA,
        ],
    )(functools.partial(_sc_interp, q_per_w=q_per_w, n_chunks=n_chunks))

    outs = []
    for s in range(NSLICE):
        sl = slice(s * ns, (s + 1) * ns)
        idx8, w48 = pl.pallas_call(
            functools.partial(_topk_block, m=m),
            grid=(ns // BN,),
            in_specs=[
                pl.BlockSpec((BN, 8), lambda i: (i, 0)),
                pl.BlockSpec((BN, 8), lambda i: (i, 0)),
                pl.BlockSpec((8, m), lambda i: (0, 0)),
                pl.BlockSpec((8, m), lambda i: (0, 0)),
            ],
            out_specs=[
                pl.BlockSpec((BN, K), lambda i: (i, 0)),
                pl.BlockSpec((BN, K * LANES), lambda i: (i, 0)),
            ],
            out_shape=[
                jax.ShapeDtypeStruct((ns, K), jnp.int32),
                jax.ShapeDtypeStruct((ns, K * LANES), jnp.float32),
            ],
        )(p_pad[sl], pb_pad[sl], vt_pad, vtb_pad)
        idx_flat = idx8.reshape(-1)
        w_exp = w48.reshape(ns * K, LANES)
        outs.append(sc(v_feats, idx_flat, w_exp))
    return jnp.concatenate(outs, axis=0)


# speedup vs baseline: 1.1282x; 1.1282x over previous
import functools

import jax
import jax.numpy as jnp
from jax import lax
from jax.experimental import pallas as pl
from jax.experimental.pallas import tpu as pltpu
from jax.experimental.pallas import tpu_sc as plsc

K = 3
BN = 512
CH = 32
LANES = 16


def _topk_block(p_ref, pb_ref, vt_ref, vtb_ref, idx_ref, w_ref, *, m):
    p = p_ref[...]
    pb = pb_ref[...]
    vt = vt_ref[...]
    vtb = vtb_ref[...]
    dot = jnp.dot(pb, vtb, preferred_element_type=jnp.float32)
    x, y, z = p[:, 0:1], p[:, 1:2], p[:, 2:3]
    p2 = x * x + y * y + z * z
    v0, v1, v2 = vt[0:1, :], vt[1:2, :], vt[2:3, :]
    vsq = v0 * v0 + v1 * v1 + v2 * v2
    pvsum = p2 + vsq
    dmat = pvsum - 2.0 * dot

    colsf = lax.broadcasted_iota(jnp.int32, dmat.shape, 1).astype(jnp.float32)
    mf = jnp.float32(m)
    inf = jnp.float32(jnp.inf)
    dw = dmat
    ikfs, ohs = [], []
    for k in range(K):
        dk = jnp.min(dw, axis=1, keepdims=True)
        ikf = jnp.min(jnp.where(dw == dk, colsf, mf), axis=1, keepdims=True)
        oh = colsf == ikf
        if k < K - 1:
            dw = jnp.where(oh, inf, dw)
        ikfs.append(ikf)
        ohs.append(oh)

    dotf = x * v0 + y * v1 + z * v2
    zero = jnp.float32(0.0)
    ddir = jnp.maximum(pvsum - 2.0 * dotf, zero)
    recips = [1.0 / (jnp.sum(jnp.where(oh, ddir, zero), axis=1, keepdims=True)
                     + 1e-8) for oh in ohs]
    norm = recips[0] + recips[1] + recips[2]
    idx_ref[...] = jnp.concatenate(
        [ikf.astype(jnp.int32) for ikf in ikfs], axis=1)
    w_ref[...] = jnp.concatenate(
        [jnp.broadcast_to(rk / norm, (p.shape[0], LANES)) for rk in recips],
        axis=1)


def _sc_interp(vf_hbm, idx_hbm, w_hbm, out_hbm, idx_v, rows_v, w_v, out_buf,
               sem, *, q_per_w, n_chunks):
    info = plsc.get_sparse_core_info()
    nc = info.num_cores
    wid = lax.axis_index("s") * nc + lax.axis_index("c")
    qbase = wid * q_per_w

    def chunk_body(c, carry):
        rbase = (qbase + c * CH) * K
        pltpu.sync_copy(idx_hbm.at[pl.ds(rbase, CH * K)], idx_v)
        pltpu.async_copy(vf_hbm.at[idx_v], rows_v, sem).wait()
        pltpu.sync_copy(w_hbm.at[pl.ds(rbase, CH * K)], w_v)

        def q_body(qi, qcarry):
            r0 = qi * K
            w0 = w_v[r0, :]
            w1 = w_v[r0 + 1, :]
            w2 = w_v[r0 + 2, :]
            for j in range(0, 256, LANES):
                seg = pl.ds(j, LANES)
                acc = w0 * rows_v[r0, seg]
                acc = acc + w1 * rows_v[r0 + 1, seg]
                acc = acc + w2 * rows_v[r0 + 2, seg]
                out_buf[qi, seg] = acc
            return qcarry

        lax.fori_loop(0, CH, q_body, 0)
        pltpu.sync_copy(out_buf, out_hbm.at[pl.ds(qbase + c * CH, CH)])
        return carry

    lax.fori_loop(0, n_chunks, chunk_body, 0)


NSLICE = 8


@jax.jit
def kernel(v_coor, v_feats, p_coor):
    n, m = p_coor.shape[0], v_coor.shape[0]
    d = v_feats.shape[1]
    p_pad = jnp.pad(p_coor, ((0, 0), (0, 5)))
    vt_pad = jnp.pad(v_coor.T, ((0, 5), (0, 0)))
    pb_pad = p_pad.astype(jnp.bfloat16)
    vtb_pad = vt_pad.astype(jnp.bfloat16)

    info = plsc.get_sparse_core_info()
    n_workers = info.num_cores * info.num_subcores
    ns = n // NSLICE
    q_per_w = ns // n_workers
    n_chunks = q_per_w // CH
    mesh = plsc.VectorSubcoreMesh(core_axis_name="c", subcore_axis_name="s")
    sc = functools.partial(
        pl.kernel,
        mesh=mesh,
        out_type=jax.ShapeDtypeStruct((ns, d), jnp.float32),
        scratch_types=[
            pltpu.VMEM((CH * K,), jnp.int32),
            pltpu.VMEM((CH * K, d), jnp.float32),
            pltpu.VMEM((CH * K, LANES), jnp.float32),
            pltpu.VMEM((CH, d), jnp.float32),
            pltpu.SemaphoreType.DMA,
        ],
    )(functools.partial(_sc_interp, q_per_w=q_per_w, n_chunks=n_chunks))

    outs = []
    for s in range(NSLICE):
        sl = slice(s * ns, (s + 1) * ns)
        idx8, w48 = pl.pallas_call(
            functools.partial(_topk_block, m=m),
            grid=(ns // BN,),
            in_specs=[
                pl.BlockSpec((BN, 8), lambda i: (i, 0)),
                pl.BlockSpec((BN, 8), lambda i: (i, 0)),
                pl.BlockSpec((8, m), lambda i: (0, 0)),
                pl.BlockSpec((8, m), lambda i: (0, 0)),
            ],
            out_specs=[
                pl.BlockSpec((BN, K), lambda i: (i, 0)),
                pl.BlockSpec((BN, K * LANES), lambda i: (i, 0)),
            ],
            out_shape=[
                jax.ShapeDtypeStruct((ns, K), jnp.int32),
                jax.ShapeDtypeStruct((ns, K * LANES), jnp.float32),
            ],
        )(p_pad[sl], pb_pad[sl], vt_pad, vtb_pad)
        idx_flat = idx8.reshape(-1)
        w_exp = w48.reshape(ns * K, LANES)
        outs.append(sc(v_feats, idx_flat, w_exp))
    return jnp.concatenate(outs, axis=0)
